# Initial kernel scaffold; baseline (speedup 1.0000x reference)
import jax
import jax.numpy as jnp
from jax.experimental import pallas as pl


def kernel(x, coords, edge_attr, bc_disp, bc_rot, params, edge_index):
    raise NotImplementedError("write your pallas kernel here")


# speedup vs baseline: 1.9544x; 1.9544x over previous
import functools

import jax
import jax.numpy as jnp
from jax import lax
from jax.experimental import pallas as pl
from jax.experimental.pallas import tpu as pltpu
from jax.experimental.pallas import tpu_sc as plsc

H = 128
NLAYERS = 6
NC = 2
NS = 16
CH = 128
RN = 1024
RE = 1024

@functools.cache
def _sc_mesh():
    return plsc.VectorSubcoreMesh(core_axis_name="c", subcore_axis_name="s",
                                  num_cores=NC, num_subcores=NS)


def _relu(v):
    return jnp.maximum(v, 0.0)


def _dot(a, b):
    return jnp.dot(a, b, preferred_element_type=jnp.float32)


def _node_enc_body(xin, w0, b0, w1, b1, ws, wd, h_o, hs_o, hd_o):
    t = _relu(_dot(xin[...], w0[...]) + b0[...])
    h = _relu(_dot(t, w1[...]) + b1[...])
    h_o[...] = h
    hs_o[...] = _dot(h, ws[...])
    hd_o[...] = _dot(h, wd[...])


def _node_encode(npad, xin, w0, b0, w1, b1, ws, wd):
    grid = (npad // RN,)
    row = lambda i: (i, 0)
    full = lambda i: (0, 0)
    return pl.pallas_call(
        _node_enc_body,
        grid=grid,
        in_specs=[
            pl.BlockSpec((RN, 16), row),
            pl.BlockSpec((16, H), full),
            pl.BlockSpec((1, H), full),
            pl.BlockSpec((H, H), full),
            pl.BlockSpec((1, H), full),
            pl.BlockSpec((H, H), full),
            pl.BlockSpec((H, H), full),
        ],
        out_specs=[pl.BlockSpec((RN, H), row)] * 3,
        out_shape=[jax.ShapeDtypeStruct((npad, H), jnp.float32)] * 3,
    )(xin, w0, b0, w1, b1, ws, wd)


def _edge_enc_body(ea, w0, b0, w1, b1, wle, ble, ep_o):
    t = _relu(_dot(ea[...], w0[...]) + b0[...])
    e2 = _relu(_dot(t, w1[...]) + b1[...])
    for l in range(NLAYERS):
        ep_o[l] = _dot(e2, wle[l]) + ble[l]


def _edge_encode(epad, ea, w0, b0, w1, b1, wle, ble):
    grid = (epad // RE,)
    return pl.pallas_call(
        _edge_enc_body,
        grid=grid,
        in_specs=[
            pl.BlockSpec((RE, 8), lambda i: (i, 0)),
            pl.BlockSpec((8, H), lambda i: (0, 0)),
            pl.BlockSpec((1, H), lambda i: (0, 0)),
            pl.BlockSpec((H, H), lambda i: (0, 0)),
            pl.BlockSpec((1, H), lambda i: (0, 0)),
            pl.BlockSpec((NLAYERS, H, H), lambda i: (0, 0, 0)),
            pl.BlockSpec((NLAYERS, 1, H), lambda i: (0, 0, 0)),
        ],
        out_specs=pl.BlockSpec((NLAYERS, RE, H), lambda i: (0, i, 0)),
        out_shape=jax.ShapeDtypeStruct((NLAYERS, epad, H), jnp.float32),
    )(ea, w0, b0, w1, b1, wle, ble)


def _post_body(has_next, h_r, p0, p1, d0, d1, w2, b2, wn1a, wn1b, bn1, wn2,
               bn2, g, b, ws, wd, *outs):
    h = h_r[...]
    aggp = p0[...] + p1[...]
    deg = d0[:, 0:1] + d1[:, 0:1]
    agg = _dot(aggp, w2[...]) + deg * b2[...]
    u = _relu(_dot(h, wn1a[...]) + _dot(agg, wn1b[...]) + bn1[...])
    u2 = _dot(u, wn2[...]) + bn2[...]
    r = h + u2
    mu = jnp.mean(r, axis=-1, keepdims=True)
    var = jnp.mean((r - mu) ** 2, axis=-1, keepdims=True)
    hn = (r - mu) * lax.rsqrt(var + 1e-5) * g[...] + b[...]
    outs[0][...] = hn
    if has_next:
        outs[1][...] = _dot(hn, ws[...])
        outs[2][...] = _dot(hn, wd[...])


def _node_update(npad, has_next, h, p0, p1, d0, d1, w2, b2, wn1a, wn1b, bn1,
                 wn2, bn2, g, b, ws, wd):
    grid = (npad // RN,)
    row = lambda i: (i, 0)
    full = lambda i: (0, 0)
    n_out = 3 if has_next else 1
    return pl.pallas_call(
        functools.partial(_post_body, has_next),
        grid=grid,
        in_specs=[
            pl.BlockSpec((RN, H), row),
            pl.BlockSpec((RN, H), row),
            pl.BlockSpec((RN, H), row),
            pl.BlockSpec((RN, 16), row),
            pl.BlockSpec((RN, 16), row),
        ] + [pl.BlockSpec((H, H), full) if w.ndim == 2 and w.shape[0] == H
             else pl.BlockSpec((1, H), full)
             for w in (w2, b2, wn1a, wn1b, bn1, wn2, bn2, g, b, ws, wd)],
        out_specs=[pl.BlockSpec((RN, H), row)] * n_out,
        out_shape=[jax.ShapeDtypeStruct((npad, H), jnp.float32)] * n_out,
    )(h, p0, p1, d0, d1, w2, b2, wn1a, wn1b, bn1, wn2, bn2, g, b, ws, wd)


def _dec_body(h_r, w1, b1, w2, b2, w3, b3, out):
    t = _relu(_dot(h_r[...], w1[...]) + b1[...])
    t = _relu(_dot(t, w2[...]) + b2[...])
    out[...] = _dot(t, w3[...]) + b3[...]


def _decode(npad, h, w1, b1, w2, b2, w3, b3):
    grid = (npad // RN,)
    row = lambda i: (i, 0)
    full = lambda i: (0, 0)
    return pl.pallas_call(
        _dec_body,
        grid=grid,
        in_specs=[
            pl.BlockSpec((RN, H), row),
            pl.BlockSpec((H, H), full),
            pl.BlockSpec((1, H), full),
            pl.BlockSpec((H, H), full),
            pl.BlockSpec((1, H), full),
            pl.BlockSpec((H, H), full),
            pl.BlockSpec((1, H), full),
        ],
        out_specs=pl.BlockSpec((RN, H), row),
        out_shape=jax.ShapeDtypeStruct((npad, H), jnp.float32),
    )(h, w1, b1, w2, b2, w3, b3)


def _sc_layer_fn(npad, epad, nsc):
    rows_per_sub = nsc // NS
    epw = epad // (NC * NS)
    nchunk = epw // CH

    @functools.partial(
        pl.kernel,
        mesh=_sc_mesh(),
        out_type=jax.ShapeDtypeStruct((NC, npad, H), jnp.float32),
        scratch_types=[
            pltpu.VMEM((CH,), jnp.int32),
            pltpu.VMEM((CH,), jnp.int32),
            pltpu.VMEM((CH, H), jnp.float32),
            pltpu.VMEM((CH, H), jnp.float32),
            pltpu.VMEM((CH, H), jnp.float32),
            pltpu.VMEM_SHARED((nsc, H), jnp.float32),
            pltpu.SemaphoreType.DMA,
            pltpu.SemaphoreType.DMA,
        ],
    )
    def sc_layer(hs_hbm, hd_hbm, ep_hbm, src_hbm, dst_hbm, out_hbm,
                 src_v, dst_v, gs_v, gd_v, ep_v, acc_sh, sem1, sem2):
        c = lax.axis_index("c")
        s = lax.axis_index("s")

        def zbody(r, carry):
            for k in range(8):
                gs_v[r, pl.ds(k * 16, 16)] = jnp.zeros((16,), jnp.float32)
            return carry
        lax.fori_loop(0, CH, zbody, 0)
        row0 = s * rows_per_sub
        left = rows_per_sub
        off = 0
        while left > 0:
            step = min(CH, left)
            pltpu.sync_copy(gs_v.at[pl.ds(0, step)],
                            acc_sh.at[pl.ds(row0 + off, step)])
            off += step
            left -= step
        plsc.subcore_barrier()

        wbase = (c * NS + s) * epw

        def chunk(i, carry):
            base = wbase + i * CH
            pltpu.sync_copy(src_hbm.at[pl.ds(base, CH)], src_v)
            pltpu.sync_copy(dst_hbm.at[pl.ds(base, CH)], dst_v)
            cp1 = pltpu.async_copy(hs_hbm.at[src_v], gs_v, sem1)
            cp2 = pltpu.async_copy(hd_hbm.at[dst_v], gd_v, sem2)
            pltpu.sync_copy(ep_hbm.at[pl.ds(base, CH)], ep_v)
            cp1.wait()
            cp2.wait()

            def ebody(e, carry2):
                for k in range(8):
                    sl = pl.ds(k * 16, 16)
                    ep_v[e, sl] = jnp.maximum(
                        gs_v[e, sl] + gd_v[e, sl] + ep_v[e, sl], 0.0)
                return carry2
            lax.fori_loop(0, CH, ebody, 0)
            pltpu.sync_copy(ep_v, acc_sh.at[dst_v], add=True)
            return carry
        lax.fori_loop(0, nchunk, chunk, 0)
        plsc.subcore_barrier()

        pltpu.sync_copy(acc_sh.at[pl.ds(row0, rows_per_sub)],
                        out_hbm.at[c, pl.ds(row0, rows_per_sub)])

    return sc_layer


def _sc_deg_fn(npad, epad):
    rows_per_sub = npad // NS
    epw = epad // (NC * NS)
    nchunk = epw // CH

    @functools.partial(
        pl.kernel,
        mesh=_sc_mesh(),
        out_type=jax.ShapeDtypeStruct((NC, npad, 16), jnp.float32),
        scratch_types=[
            pltpu.VMEM((CH,), jnp.int32),
            pltpu.VMEM((CH, 16), jnp.float32),
            pltpu.VMEM_SHARED((npad, 16), jnp.float32),
        ],
    )
    def sc_deg(dst_hbm, out_hbm, dst_v, ones_v, acc_sh):
        c = lax.axis_index("c")
        s = lax.axis_index("s")

        def zbody(r, carry):
            ones_v[r, pl.ds(0, 16)] = jnp.zeros((16,), jnp.float32)
            return carry
        lax.fori_loop(0, CH, zbody, 0)
        row0 = s * rows_per_sub
        for j in range(rows_per_sub // CH):
            pltpu.sync_copy(ones_v, acc_sh.at[pl.ds(row0 + j * CH, CH)])

        def obody(r, carry):
            ones_v[r, pl.ds(0, 16)] = jnp.ones((16,), jnp.float32)
            return carry
        lax.fori_loop(0, CH, obody, 0)
        plsc.subcore_barrier()

        wbase = (c * NS + s) * epw

        def chunk(i, carry):
            base = wbase + i * CH
            pltpu.sync_copy(dst_hbm.at[pl.ds(base, CH)], dst_v)
            pltpu.sync_copy(ones_v, acc_sh.at[dst_v], add=True)
            return carry
        lax.fori_loop(0, nchunk, chunk, 0)
        plsc.subcore_barrier()

        for j in range(rows_per_sub // CH):
            r0 = row0 + j * CH
            pltpu.sync_copy(acc_sh.at[pl.ds(r0, CH)],
                            out_hbm.at[c, pl.ds(r0, CH)])

    return sc_deg


def _pad_to(v, m):
    return ((v + m - 1) // m) * m


def kernel(x, coords, edge_attr, bc_disp, bc_rot, params, edge_index):
    n = x.shape[0]
    e = edge_index.shape[1]
    npad = _pad_to(n, NS * CH)
    epad = _pad_to(e, NC * NS * CH)
    nsc = _pad_to(n + 8, NS * 8)

    xin = jnp.concatenate([coords, x[:, 3:]], axis=-1)
    xin = jnp.pad(xin, ((0, npad - n), (0, 16 - xin.shape[1])))
    ea = jnp.pad(edge_attr, ((0, epad - e), (0, 8 - edge_attr.shape[1])))
    pad_idx = n
    src = jnp.pad(edge_index[0], (0, epad - e), constant_values=pad_idx)
    dst = jnp.pad(edge_index[1], (0, epad - e), constant_values=pad_idx)

    def w_of(p):
        return p["W"]

    def b_of(p):
        return p["b"].reshape(1, -1)

    pr = params
    ne0, ne1 = pr["node_enc"]
    ee0, ee1 = pr["edge_enc"]
    layers = pr["layers"]

    w_ne0 = jnp.pad(w_of(ne0), ((0, 16 - 9), (0, 0)))
    w_ee0 = jnp.pad(w_of(ee0), ((0, 8 - 7), (0, 0)))

    wls = [w_of(l["msg1"])[0:H] for l in layers]
    wld = [w_of(l["msg1"])[H:2 * H] for l in layers]
    wle = jnp.stack([w_of(l["msg1"])[2 * H:] for l in layers])
    ble = jnp.stack([l["msg1"]["b"].reshape(1, H) for l in layers])

    h, hs, hd = _node_encode(npad, xin, w_ne0, b_of(ne0), w_of(ne1),
                             b_of(ne1), wls[0], wld[0])
    ep_all = _edge_encode(epad, ea, w_ee0, b_of(ee0), w_of(ee1), b_of(ee1),
                          wle, ble)

    sc_layer = _sc_layer_fn(npad, epad, nsc)
    deg_parts = _sc_deg_fn(npad, epad)(dst)
    d0, d1 = deg_parts[0], deg_parts[1]

    for l in range(NLAYERS):
        lp = layers[l]
        parts = sc_layer(hs, hd, ep_all[l], src, dst)
        has_next = l + 1 < NLAYERS
        nxt = layers[l + 1] if has_next else layers[0]
        outs = _node_update(
            npad, has_next, h, parts[0], parts[1], d0, d1,
            w_of(lp["msg2"]), b_of(lp["msg2"]),
            w_of(lp["node1"])[0:H], w_of(lp["node1"])[H:2 * H],
            b_of(lp["node1"]), w_of(lp["node2"]), b_of(lp["node2"]),
            lp["ln_g"].reshape(1, H), lp["ln_b"].reshape(1, H),
            wls[(l + 1) % NLAYERS], wld[(l + 1) % NLAYERS])
        if has_next:
            h, hs, hd = outs
        else:
            h = outs[0]

    d0p, d1p, d2p = pr["dec"]
    w_d2 = jnp.pad(w_of(d1p), ((0, 0), (0, H - 64)))
    b_d2 = jnp.pad(b_of(d1p), ((0, 0), (0, H - 64)))
    w_d3 = jnp.pad(w_of(d2p), ((0, H - 64), (0, H - 3)))
    b_d3 = jnp.pad(b_of(d2p), ((0, 0), (0, H - 3)))
    out = _decode(npad, h, w_of(d0p), b_of(d0p), w_d2, b_d2, w_d3, b_d3)

    pred = out[:n, :3]
    mask = jnp.concatenate([1.0 - bc_disp, 1.0 - bc_disp, 1.0 - bc_rot],
                           axis=-1)
    return pred * mask


# speedup vs baseline: 2.4026x; 1.2293x over previous
import functools

import jax
import jax.numpy as jnp
from jax import lax
from jax.experimental import pallas as pl
from jax.experimental.pallas import tpu as pltpu
from jax.experimental.pallas import tpu_sc as plsc

H = 128
NLAYERS = 6
NC = 2
NS = 16
CH = 64
RN = 1024
RE = 1024

@functools.cache
def _sc_mesh():
    return plsc.VectorSubcoreMesh(core_axis_name="c", subcore_axis_name="s",
                                  num_cores=NC, num_subcores=NS)


def _relu(v):
    return jnp.maximum(v, 0.0)


def _dot(a, b):
    return jnp.dot(a, b, preferred_element_type=jnp.float32)


def _node_enc_body(xin, w0, b0, w1, b1, ws, wd, h_o, hs_o, hd_o):
    t = _relu(_dot(xin[...], w0[...]) + b0[...])
    h = _relu(_dot(t, w1[...]) + b1[...])
    h_o[...] = h
    hs_o[...] = _dot(h, ws[...])
    hd_o[...] = _dot(h, wd[...])


def _node_encode(npad, xin, w0, b0, w1, b1, ws, wd):
    grid = (npad // RN,)
    row = lambda i: (i, 0)
    full = lambda i: (0, 0)
    return pl.pallas_call(
        _node_enc_body,
        grid=grid,
        in_specs=[
            pl.BlockSpec((RN, 16), row),
            pl.BlockSpec((16, H), full),
            pl.BlockSpec((1, H), full),
            pl.BlockSpec((H, H), full),
            pl.BlockSpec((1, H), full),
            pl.BlockSpec((H, H), full),
            pl.BlockSpec((H, H), full),
        ],
        out_specs=[pl.BlockSpec((RN, H), row)] * 3,
        out_shape=[jax.ShapeDtypeStruct((npad, H), jnp.float32)] * 3,
    )(xin, w0, b0, w1, b1, ws, wd)


def _edge_enc_body(ea, w0, b0, w1, b1, wle, ble, ep_o):
    t = _relu(_dot(ea[...], w0[...]) + b0[...])
    e2 = _relu(_dot(t, w1[...]) + b1[...])
    for l in range(NLAYERS):
        ep_o[l] = _dot(e2, wle[l]) + ble[l]


def _edge_encode(epad, ea, w0, b0, w1, b1, wle, ble):
    grid = (epad // RE,)
    return pl.pallas_call(
        _edge_enc_body,
        grid=grid,
        in_specs=[
            pl.BlockSpec((RE, 8), lambda i: (i, 0)),
            pl.BlockSpec((8, H), lambda i: (0, 0)),
            pl.BlockSpec((1, H), lambda i: (0, 0)),
            pl.BlockSpec((H, H), lambda i: (0, 0)),
            pl.BlockSpec((1, H), lambda i: (0, 0)),
            pl.BlockSpec((NLAYERS, H, H), lambda i: (0, 0, 0)),
            pl.BlockSpec((NLAYERS, 1, H), lambda i: (0, 0, 0)),
        ],
        out_specs=pl.BlockSpec((NLAYERS, RE, H), lambda i: (0, i, 0)),
        out_shape=jax.ShapeDtypeStruct((NLAYERS, epad, H), jnp.float32),
    )(ea, w0, b0, w1, b1, wle, ble)


def _post_body(has_next, h_r, p0, p1, d0, d1, w2, b2, wn1a, wn1b, bn1, wn2,
               bn2, g, b, ws, wd, *outs):
    h = h_r[...]
    aggp = p0[...] + p1[...]
    deg = d0[:, 0:1] + d1[:, 0:1]
    agg = _dot(aggp, w2[...]) + deg * b2[...]
    u = _relu(_dot(h, wn1a[...]) + _dot(agg, wn1b[...]) + bn1[...])
    u2 = _dot(u, wn2[...]) + bn2[...]
    r = h + u2
    mu = jnp.mean(r, axis=-1, keepdims=True)
    var = jnp.mean((r - mu) ** 2, axis=-1, keepdims=True)
    hn = (r - mu) * lax.rsqrt(var + 1e-5) * g[...] + b[...]
    outs[0][...] = hn
    if has_next:
        outs[1][...] = _dot(hn, ws[...])
        outs[2][...] = _dot(hn, wd[...])


def _node_update(npad, has_next, h, p0, p1, d0, d1, w2, b2, wn1a, wn1b, bn1,
                 wn2, bn2, g, b, ws, wd):
    grid = (npad // RN,)
    row = lambda i: (i, 0)
    full = lambda i: (0, 0)
    n_out = 3 if has_next else 1
    return pl.pallas_call(
        functools.partial(_post_body, has_next),
        grid=grid,
        in_specs=[
            pl.BlockSpec((RN, H), row),
            pl.BlockSpec((RN, H), row),
            pl.BlockSpec((RN, H), row),
            pl.BlockSpec((RN, 16), row),
            pl.BlockSpec((RN, 16), row),
        ] + [pl.BlockSpec((H, H), full) if w.ndim == 2 and w.shape[0] == H
             else pl.BlockSpec((1, H), full)
             for w in (w2, b2, wn1a, wn1b, bn1, wn2, bn2, g, b, ws, wd)],
        out_specs=[pl.BlockSpec((RN, H), row)] * n_out,
        out_shape=[jax.ShapeDtypeStruct((npad, H), jnp.float32)] * n_out,
    )(h, p0, p1, d0, d1, w2, b2, wn1a, wn1b, bn1, wn2, bn2, g, b, ws, wd)


def _dec_body(h_r, w1, b1, w2, b2, w3, b3, out):
    t = _relu(_dot(h_r[...], w1[...]) + b1[...])
    t = _relu(_dot(t, w2[...]) + b2[...])
    out[...] = _dot(t, w3[...]) + b3[...]


def _decode(npad, h, w1, b1, w2, b2, w3, b3):
    grid = (npad // RN,)
    row = lambda i: (i, 0)
    full = lambda i: (0, 0)
    return pl.pallas_call(
        _dec_body,
        grid=grid,
        in_specs=[
            pl.BlockSpec((RN, H), row),
            pl.BlockSpec((H, H), full),
            pl.BlockSpec((1, H), full),
            pl.BlockSpec((H, H), full),
            pl.BlockSpec((1, H), full),
            pl.BlockSpec((H, H), full),
            pl.BlockSpec((1, H), full),
        ],
        out_specs=pl.BlockSpec((RN, H), row),
        out_shape=jax.ShapeDtypeStruct((npad, H), jnp.float32),
    )(h, w1, b1, w2, b2, w3, b3)


def _sc_layer_fn(npad, epad, nsc):
    rows_per_sub = nsc // NS
    epw = epad // (NC * NS)
    nchunk = epw // CH

    @functools.partial(
        pl.kernel,
        mesh=_sc_mesh(),
        out_type=jax.ShapeDtypeStruct((NC, npad, H), jnp.float32),
        scratch_types=(
            [pltpu.VMEM((CH,), jnp.int32)] * 4
            + [pltpu.VMEM((CH, H), jnp.float32)] * 6
            + [pltpu.VMEM_SHARED((nsc, H), jnp.float32)]
            + [pltpu.SemaphoreType.DMA] * 8
        ),
    )
    def sc_layer(hs_hbm, hd_hbm, ep_hbm, src_hbm, dst_hbm, out_hbm,
                 src0, src1, dst0, dst1, gs0, gs1, gd0, gd1, ep0, ep1,
                 acc_sh, *sems):
        c = lax.axis_index("c")
        s = lax.axis_index("s")
        wid = c * NS + s
        wbase = wid * epw
        srcs, dsts = (src0, src1), (dst0, dst1)
        gss, gds, eps = (gs0, gs1), (gd0, gd1), (ep0, ep1)
        gsem = (sems[0:3], sems[3:6])
        isem = (sems[6], sems[7])

        def zbody(r, carry):
            for k in range(8):
                gs0[r, pl.ds(k * 16, 16)] = jnp.zeros((16,), jnp.float32)
            return carry
        lax.fori_loop(0, CH, zbody, 0)
        row0 = s * rows_per_sub
        left = rows_per_sub
        off = 0
        while left > 0:
            step = min(CH, left)
            pltpu.sync_copy(gs0.at[pl.ds(0, step)],
                            acc_sh.at[pl.ds(row0 + off, step)])
            off += step
            left -= step
        plsc.subcore_barrier()

        def issue_idx(i, b):
            base = wbase + i * CH
            pltpu.async_copy(src_hbm.at[pl.ds(base, CH)], srcs[b], isem[b])
            pltpu.async_copy(dst_hbm.at[pl.ds(base, CH)], dsts[b], isem[b])

        def wait_idx(b):
            pltpu.make_async_copy(src_hbm.at[pl.ds(0, CH)], srcs[b],
                                  isem[b]).wait()
            pltpu.make_async_copy(dst_hbm.at[pl.ds(0, CH)], dsts[b],
                                  isem[b]).wait()

        def start_gathers(i, b):
            pltpu.async_copy(hs_hbm.at[srcs[b]], gss[b], gsem[b][0])
            pltpu.async_copy(hd_hbm.at[dsts[b]], gds[b], gsem[b][1])
            pltpu.async_copy(ep_hbm.at[pl.ds(wbase + i * CH, CH)], eps[b],
                             gsem[b][2])

        def wait_gathers(b):
            pltpu.make_async_copy(hs_hbm.at[srcs[b]], gss[b],
                                  gsem[b][0]).wait()
            pltpu.make_async_copy(hd_hbm.at[dsts[b]], gds[b],
                                  gsem[b][1]).wait()
            pltpu.make_async_copy(ep_hbm.at[pl.ds(0, CH)], eps[b],
                                  gsem[b][2]).wait()

        def process(b):
            gs_v, gd_v, ep_v = gss[b], gds[b], eps[b]

            def ebody(e2, carry2):
                for k in range(8):
                    sl = pl.ds(k * 16, 16)
                    ep_v[e2, sl] = jnp.maximum(
                        gs_v[e2, sl] + gd_v[e2, sl] + ep_v[e2, sl], 0.0)
                return carry2
            lax.fori_loop(0, CH, ebody, 0)
            pltpu.sync_copy(ep_v, acc_sh.at[dsts[b]], add=True)

        pltpu.sync_copy(src_hbm.at[pl.ds(wbase, CH)], src0)
        pltpu.sync_copy(dst_hbm.at[pl.ds(wbase, CH)], dst0)
        start_gathers(0, 0)
        issue_idx(1, 1)

        def pair(g, carry):
            i = 2 * g
            wait_idx(1)
            start_gathers(i + 1, 1)
            wait_gathers(0)
            process(0)
            nxt = lax.rem(i + 2, nchunk)
            pltpu.async_copy(src_hbm.at[pl.ds(wbase + nxt * CH, CH)], src0,
                             isem[0])
            pltpu.async_copy(dst_hbm.at[pl.ds(wbase + nxt * CH, CH)], dst0,
                             isem[0])
            wait_idx(0)
            start_gathers(nxt, 0)
            wait_gathers(1)
            process(1)
            nxt2 = lax.rem(i + 3, nchunk)
            issue_idx(nxt2, 1)
            return carry
        lax.fori_loop(0, nchunk // 2, pair, 0)
        wait_gathers(0)
        wait_idx(1)
        plsc.subcore_barrier()

        pltpu.sync_copy(acc_sh.at[pl.ds(row0, rows_per_sub)],
                        out_hbm.at[c, pl.ds(row0, rows_per_sub)])

    return sc_layer


def _sc_deg_fn(npad, epad):
    rows_per_sub = npad // NS
    epw = epad // (NC * NS)
    nchunk = epw // CH

    @functools.partial(
        pl.kernel,
        mesh=_sc_mesh(),
        out_type=jax.ShapeDtypeStruct((NC, npad, 16), jnp.float32),
        scratch_types=[
            pltpu.VMEM((CH,), jnp.int32),
            pltpu.VMEM((CH, 16), jnp.float32),
            pltpu.VMEM_SHARED((npad, 16), jnp.float32),
        ],
    )
    def sc_deg(dst_hbm, out_hbm, dst_v, ones_v, acc_sh):
        c = lax.axis_index("c")
        s = lax.axis_index("s")
        wbase = (c * NS + s) * epw

        def zbody(r, carry):
            ones_v[r, pl.ds(0, 16)] = jnp.zeros((16,), jnp.float32)
            return carry
        lax.fori_loop(0, CH, zbody, 0)
        row0 = s * rows_per_sub
        for j in range(rows_per_sub // CH):
            pltpu.sync_copy(ones_v, acc_sh.at[pl.ds(row0 + j * CH, CH)])

        def obody(r, carry):
            ones_v[r, pl.ds(0, 16)] = jnp.ones((16,), jnp.float32)
            return carry
        lax.fori_loop(0, CH, obody, 0)
        plsc.subcore_barrier()

        def chunk(i, carry):
            pltpu.sync_copy(dst_hbm.at[pl.ds(wbase + i * CH, CH)], dst_v)
            pltpu.sync_copy(ones_v, acc_sh.at[dst_v], add=True)
            return carry
        lax.fori_loop(0, nchunk, chunk, 0)
        plsc.subcore_barrier()

        for j in range(rows_per_sub // CH):
            r0 = row0 + j * CH
            pltpu.sync_copy(acc_sh.at[pl.ds(r0, CH)],
                            out_hbm.at[c, pl.ds(r0, CH)])

    return sc_deg


def _pad_to(v, m):
    return ((v + m - 1) // m) * m


def kernel(x, coords, edge_attr, bc_disp, bc_rot, params, edge_index):
    n = x.shape[0]
    e = edge_index.shape[1]
    npad = _pad_to(n, NS * CH)
    epad = _pad_to(e, NC * NS * CH * 2)
    nsc = _pad_to(n + 8, NS * 8)

    xin = jnp.concatenate([coords, x[:, 3:]], axis=-1)
    xin = jnp.pad(xin, ((0, npad - n), (0, 16 - xin.shape[1])))
    ea = jnp.pad(edge_attr, ((0, epad - e), (0, 8 - edge_attr.shape[1])))
    pad_idx = n
    src = jnp.pad(edge_index[0], (0, epad - e), constant_values=pad_idx)
    dst = jnp.pad(edge_index[1], (0, epad - e), constant_values=pad_idx)

    def w_of(p):
        return p["W"]

    def b_of(p):
        return p["b"].reshape(1, -1)

    pr = params
    ne0, ne1 = pr["node_enc"]
    ee0, ee1 = pr["edge_enc"]
    layers = pr["layers"]

    w_ne0 = jnp.pad(w_of(ne0), ((0, 16 - 9), (0, 0)))
    w_ee0 = jnp.pad(w_of(ee0), ((0, 8 - 7), (0, 0)))

    wls = [w_of(l["msg1"])[0:H] for l in layers]
    wld = [w_of(l["msg1"])[H:2 * H] for l in layers]
    wle = jnp.stack([w_of(l["msg1"])[2 * H:] for l in layers])
    ble = jnp.stack([l["msg1"]["b"].reshape(1, H) for l in layers])

    h, hs, hd = _node_encode(npad, xin, w_ne0, b_of(ne0), w_of(ne1),
                             b_of(ne1), wls[0], wld[0])
    ep_all = _edge_encode(epad, ea, w_ee0, b_of(ee0), w_of(ee1), b_of(ee1),
                          wle, ble)

    sc_layer = _sc_layer_fn(npad, epad, nsc)
    deg_parts = _sc_deg_fn(npad, epad)(dst)
    d0, d1 = deg_parts[0], deg_parts[1]

    for l in range(NLAYERS):
        lp = layers[l]
        parts = sc_layer(hs, hd, ep_all[l], src, dst)
        has_next = l + 1 < NLAYERS
        nxt = layers[l + 1] if has_next else layers[0]
        outs = _node_update(
            npad, has_next, h, parts[0], parts[1], d0, d1,
            w_of(lp["msg2"]), b_of(lp["msg2"]),
            w_of(lp["node1"])[0:H], w_of(lp["node1"])[H:2 * H],
            b_of(lp["node1"]), w_of(lp["node2"]), b_of(lp["node2"]),
            lp["ln_g"].reshape(1, H), lp["ln_b"].reshape(1, H),
            wls[(l + 1) % NLAYERS], wld[(l + 1) % NLAYERS])
        if has_next:
            h, hs, hd = outs
        else:
            h = outs[0]

    d0p, d1p, d2p = pr["dec"]
    w_d2 = jnp.pad(w_of(d1p), ((0, 0), (0, H - 64)))
    b_d2 = jnp.pad(b_of(d1p), ((0, 0), (0, H - 64)))
    w_d3 = jnp.pad(w_of(d2p), ((0, H - 64), (0, H - 3)))
    b_d3 = jnp.pad(b_of(d2p), ((0, 0), (0, H - 3)))
    out = _decode(npad, h, w_of(d0p), b_of(d0p), w_d2, b_d2, w_d3, b_d3)

    pred = out[:n, :3]
    mask = jnp.concatenate([1.0 - bc_disp, 1.0 - bc_disp, 1.0 - bc_rot],
                           axis=-1)
    return pred * mask


# speedup vs baseline: 3.8639x; 1.6083x over previous
import functools

import jax
import jax.numpy as jnp
from jax import lax
from jax.experimental import pallas as pl
from jax.experimental.pallas import tpu as pltpu
from jax.experimental.pallas import tpu_sc as plsc

H = 128
NLAYERS = 6
NC = 2
NS = 16
CH = 64
RN = 1024
RE = 1024

@functools.cache
def _sc_mesh():
    return plsc.VectorSubcoreMesh(core_axis_name="c", subcore_axis_name="s",
                                  num_cores=NC, num_subcores=NS)


def _relu(v):
    return jnp.maximum(v, 0.0)


def _dot(a, b):
    return jnp.dot(a, b, preferred_element_type=jnp.float32)


def _node_enc_body(xin, w0, b0, w1, b1, ws, wd, h_o, hs_o, hd_o):
    t = _relu(_dot(xin[...], w0[...]) + b0[...])
    h = _relu(_dot(t, w1[...]) + b1[...])
    h_o[...] = h
    hs_o[...] = _dot(h, ws[...])
    hd_o[...] = _dot(h, wd[...])


def _node_encode(npad, xin, w0, b0, w1, b1, ws, wd):
    grid = (npad // RN,)
    row = lambda i: (i, 0)
    full = lambda i: (0, 0)
    return pl.pallas_call(
        _node_enc_body,
        grid=grid,
        in_specs=[
            pl.BlockSpec((RN, 16), row),
            pl.BlockSpec((16, H), full),
            pl.BlockSpec((1, H), full),
            pl.BlockSpec((H, H), full),
            pl.BlockSpec((1, H), full),
            pl.BlockSpec((H, H), full),
            pl.BlockSpec((H, H), full),
        ],
        out_specs=[pl.BlockSpec((RN, H), row)] * 3,
        out_shape=[jax.ShapeDtypeStruct((npad, H), jnp.float32)] * 3,
    )(xin, w0, b0, w1, b1, ws, wd)


def _edge_enc_body(ea, w0, b0, w1, b1, wle, ble, ep_o):
    t = _relu(_dot(ea[...], w0[...]) + b0[...])
    e2 = _relu(_dot(t, w1[...]) + b1[...])
    for l in range(NLAYERS):
        ep_o[l] = _dot(e2, wle[l]) + ble[l]


def _edge_encode(epad, ea, w0, b0, w1, b1, wle, ble):
    grid = (epad // RE,)
    return pl.pallas_call(
        _edge_enc_body,
        grid=grid,
        in_specs=[
            pl.BlockSpec((RE, 8), lambda i: (i, 0)),
            pl.BlockSpec((8, H), lambda i: (0, 0)),
            pl.BlockSpec((1, H), lambda i: (0, 0)),
            pl.BlockSpec((H, H), lambda i: (0, 0)),
            pl.BlockSpec((1, H), lambda i: (0, 0)),
            pl.BlockSpec((NLAYERS, H, H), lambda i: (0, 0, 0)),
            pl.BlockSpec((NLAYERS, 1, H), lambda i: (0, 0, 0)),
        ],
        out_specs=pl.BlockSpec((NLAYERS, RE, H), lambda i: (0, i, 0)),
        out_shape=jax.ShapeDtypeStruct((NLAYERS, epad, H), jnp.float32),
    )(ea, w0, b0, w1, b1, wle, ble)


def _post_body(has_next, h_r, p0, p1, d0, d1, w2, b2, wn1a, wn1b, bn1, wn2,
               bn2, g, b, ws, wd, *outs):
    h = h_r[...]
    aggp = p0[...] + p1[...]
    deg = d0[:, 0:1] + d1[:, 0:1]
    agg = _dot(aggp, w2[...]) + deg * b2[...]
    u = _relu(_dot(h, wn1a[...]) + _dot(agg, wn1b[...]) + bn1[...])
    u2 = _dot(u, wn2[...]) + bn2[...]
    r = h + u2
    mu = jnp.mean(r, axis=-1, keepdims=True)
    var = jnp.mean((r - mu) ** 2, axis=-1, keepdims=True)
    hn = (r - mu) * lax.rsqrt(var + 1e-5) * g[...] + b[...]
    outs[0][...] = hn
    if has_next:
        outs[1][...] = _dot(hn, ws[...])
        outs[2][...] = _dot(hn, wd[...])


def _node_update(npad, has_next, h, p0, p1, d0, d1, w2, b2, wn1a, wn1b, bn1,
                 wn2, bn2, g, b, ws, wd):
    grid = (npad // RN,)
    row = lambda i: (i, 0)
    full = lambda i: (0, 0)
    n_out = 3 if has_next else 1
    return pl.pallas_call(
        functools.partial(_post_body, has_next),
        grid=grid,
        in_specs=[
            pl.BlockSpec((RN, H), row),
            pl.BlockSpec((RN, H), row),
            pl.BlockSpec((RN, H), row),
            pl.BlockSpec((RN, 16), row),
            pl.BlockSpec((RN, 16), row),
        ] + [pl.BlockSpec((H, H), full) if w.ndim == 2 and w.shape[0] == H
             else pl.BlockSpec((1, H), full)
             for w in (w2, b2, wn1a, wn1b, bn1, wn2, bn2, g, b, ws, wd)],
        out_specs=[pl.BlockSpec((RN, H), row)] * n_out,
        out_shape=[jax.ShapeDtypeStruct((npad, H), jnp.float32)] * n_out,
    )(h, p0, p1, d0, d1, w2, b2, wn1a, wn1b, bn1, wn2, bn2, g, b, ws, wd)


def _dec_body(h_r, w1, b1, w2, b2, w3, b3, out):
    t = _relu(_dot(h_r[...], w1[...]) + b1[...])
    t = _relu(_dot(t, w2[...]) + b2[...])
    out[...] = _dot(t, w3[...]) + b3[...]


def _decode(npad, h, w1, b1, w2, b2, w3, b3):
    grid = (npad // RN,)
    row = lambda i: (i, 0)
    full = lambda i: (0, 0)
    return pl.pallas_call(
        _dec_body,
        grid=grid,
        in_specs=[
            pl.BlockSpec((RN, H), row),
            pl.BlockSpec((H, H), full),
            pl.BlockSpec((1, H), full),
            pl.BlockSpec((H, H), full),
            pl.BlockSpec((1, H), full),
            pl.BlockSpec((H, H), full),
            pl.BlockSpec((1, H), full),
        ],
        out_specs=pl.BlockSpec((RN, H), row),
        out_shape=jax.ShapeDtypeStruct((npad, H), jnp.float32),
    )(h, w1, b1, w2, b2, w3, b3)


def _sc_layer_fn(npad, epad, nsc):
    rows_per_sub = nsc // NS
    epw = epad // (NC * NS)
    nchunk = epw // CH

    @functools.partial(
        pl.kernel,
        mesh=_sc_mesh(),
        out_type=jax.ShapeDtypeStruct((NC, npad, H), jnp.float32),
        scratch_types=(
            [pltpu.VMEM((CH,), jnp.int32)] * 4
            + [pltpu.VMEM((CH, H), jnp.float32)] * 6
            + [pltpu.VMEM_SHARED((nsc, H), jnp.float32)]
            + [pltpu.SemaphoreType.DMA] * 8
        ),
    )
    def sc_layer(hs_hbm, hd_hbm, ep_hbm, src_hbm, dst_hbm, out_hbm,
                 src0, src1, dst0, dst1, gs0, gs1, gd0, gd1, ep0, ep1,
                 acc_sh, *sems):
        c = lax.axis_index("c")
        s = lax.axis_index("s")
        wid = c * NS + s
        wbase = wid * epw
        srcs, dsts = (src0, src1), (dst0, dst1)
        gss, gds, eps = (gs0, gs1), (gd0, gd1), (ep0, ep1)
        gsem = (sems[0:3], sems[3:6])
        isem = (sems[6], sems[7])

        def zbody(r, carry):
            for k in range(8):
                gs0[r, pl.ds(k * 16, 16)] = jnp.zeros((16,), jnp.float32)
            return carry
        lax.fori_loop(0, CH, zbody, 0)
        row0 = s * rows_per_sub
        left = rows_per_sub
        off = 0
        while left > 0:
            step = min(CH, left)
            pltpu.sync_copy(gs0.at[pl.ds(0, step)],
                            acc_sh.at[pl.ds(row0 + off, step)])
            off += step
            left -= step
        plsc.subcore_barrier()

        def issue_idx(i, b):
            base = wbase + i * CH
            pltpu.async_copy(src_hbm.at[pl.ds(base, CH)], srcs[b], isem[b])
            pltpu.async_copy(dst_hbm.at[pl.ds(base, CH)], dsts[b], isem[b])

        def wait_idx(b):
            pltpu.make_async_copy(src_hbm.at[pl.ds(0, CH)], srcs[b],
                                  isem[b]).wait()
            pltpu.make_async_copy(dst_hbm.at[pl.ds(0, CH)], dsts[b],
                                  isem[b]).wait()

        def start_gathers(i, b):
            pltpu.async_copy(hs_hbm.at[srcs[b]], gss[b], gsem[b][0])
            pltpu.async_copy(hd_hbm.at[dsts[b]], gds[b], gsem[b][1])
            pltpu.async_copy(ep_hbm.at[pl.ds(wbase + i * CH, CH)], eps[b],
                             gsem[b][2])

        def wait_gathers(b):
            pltpu.make_async_copy(hs_hbm.at[srcs[b]], gss[b],
                                  gsem[b][0]).wait()
            pltpu.make_async_copy(hd_hbm.at[dsts[b]], gds[b],
                                  gsem[b][1]).wait()
            pltpu.make_async_copy(ep_hbm.at[pl.ds(0, CH)], eps[b],
                                  gsem[b][2]).wait()

        def process(b):
            gs_v, gd_v, ep_v = gss[b], gds[b], eps[b]

            def ebody(e2, carry2):
                for k in range(8):
                    sl = pl.ds(k * 16, 16)
                    ep_v[e2, sl] = jnp.maximum(
                        gs_v[e2, sl] + gd_v[e2, sl] + ep_v[e2, sl], 0.0)
                return carry2
            lax.fori_loop(0, CH, ebody, 0)
            pltpu.sync_copy(ep_v, acc_sh.at[dsts[b]], add=True)

        pltpu.sync_copy(src_hbm.at[pl.ds(wbase, CH)], src0)
        pltpu.sync_copy(dst_hbm.at[pl.ds(wbase, CH)], dst0)
        start_gathers(0, 0)
        issue_idx(1, 1)

        def pair(g, carry):
            i = 2 * g
            wait_idx(1)
            start_gathers(i + 1, 1)
            wait_gathers(0)
            process(0)
            nxt = lax.rem(i + 2, nchunk)
            pltpu.async_copy(src_hbm.at[pl.ds(wbase + nxt * CH, CH)], src0,
                             isem[0])
            pltpu.async_copy(dst_hbm.at[pl.ds(wbase + nxt * CH, CH)], dst0,
                             isem[0])
            wait_idx(0)
            start_gathers(nxt, 0)
            wait_gathers(1)
            process(1)
            nxt2 = lax.rem(i + 3, nchunk)
            issue_idx(nxt2, 1)
            return carry
        lax.fori_loop(0, nchunk // 2, pair, 0)
        wait_gathers(0)
        wait_idx(1)
        plsc.subcore_barrier()

        pltpu.sync_copy(acc_sh.at[pl.ds(row0, rows_per_sub)],
                        out_hbm.at[c, pl.ds(row0, rows_per_sub)])

    return sc_layer


def _sc_deg_fn(npad, epad):
    rows_per_sub = npad // NS
    epw = epad // (NC * NS)
    nchunk = epw // CH

    @functools.partial(
        pl.kernel,
        mesh=_sc_mesh(),
        out_type=jax.ShapeDtypeStruct((NC, npad, 16), jnp.float32),
        scratch_types=[
            pltpu.VMEM((CH,), jnp.int32),
            pltpu.VMEM((CH, 16), jnp.float32),
            pltpu.VMEM_SHARED((npad, 16), jnp.float32),
        ],
    )
    def sc_deg(dst_hbm, out_hbm, dst_v, ones_v, acc_sh):
        c = lax.axis_index("c")
        s = lax.axis_index("s")
        wbase = (c * NS + s) * epw

        def zbody(r, carry):
            ones_v[r, pl.ds(0, 16)] = jnp.zeros((16,), jnp.float32)
            return carry
        lax.fori_loop(0, CH, zbody, 0)
        row0 = s * rows_per_sub
        for j in range(rows_per_sub // CH):
            pltpu.sync_copy(ones_v, acc_sh.at[pl.ds(row0 + j * CH, CH)])

        def obody(r, carry):
            ones_v[r, pl.ds(0, 16)] = jnp.ones((16,), jnp.float32)
            return carry
        lax.fori_loop(0, CH, obody, 0)
        plsc.subcore_barrier()

        def chunk(i, carry):
            pltpu.sync_copy(dst_hbm.at[pl.ds(wbase + i * CH, CH)], dst_v)
            pltpu.sync_copy(ones_v, acc_sh.at[dst_v], add=True)
            return carry
        lax.fori_loop(0, nchunk, chunk, 0)
        plsc.subcore_barrier()

        for j in range(rows_per_sub // CH):
            r0 = row0 + j * CH
            pltpu.sync_copy(acc_sh.at[pl.ds(r0, CH)],
                            out_hbm.at[c, pl.ds(r0, CH)])

    return sc_deg


def _pad_to(v, m):
    return ((v + m - 1) // m) * m


def kernel(x, coords, edge_attr, bc_disp, bc_rot, params, edge_index):
    n = x.shape[0]
    e = edge_index.shape[1]
    npad = _pad_to(n, NS * CH)
    epad = _pad_to(e, NC * NS * CH * 2)
    nsc = _pad_to(n + 64, NS * 8)

    xin = jnp.concatenate([coords, x[:, 3:]], axis=-1)
    xin = jnp.pad(xin, ((0, npad - n), (0, 16 - xin.shape[1])))
    ea = jnp.pad(edge_attr, ((0, epad - e), (0, 8 - edge_attr.shape[1])))
    pad_idx = n + jnp.arange(epad - e, dtype=jnp.int32) % 64
    src = jnp.concatenate([edge_index[0], pad_idx])
    dst = jnp.concatenate([edge_index[1], pad_idx])

    def w_of(p):
        return p["W"]

    def b_of(p):
        return p["b"].reshape(1, -1)

    pr = params
    ne0, ne1 = pr["node_enc"]
    ee0, ee1 = pr["edge_enc"]
    layers = pr["layers"]

    w_ne0 = jnp.pad(w_of(ne0), ((0, 16 - 9), (0, 0)))
    w_ee0 = jnp.pad(w_of(ee0), ((0, 8 - 7), (0, 0)))

    wls = [w_of(l["msg1"])[0:H] for l in layers]
    wld = [w_of(l["msg1"])[H:2 * H] for l in layers]
    wle = jnp.stack([w_of(l["msg1"])[2 * H:] for l in layers])
    ble = jnp.stack([l["msg1"]["b"].reshape(1, H) for l in layers])

    h, hs, hd = _node_encode(npad, xin, w_ne0, b_of(ne0), w_of(ne1),
                             b_of(ne1), wls[0], wld[0])
    ep_all = _edge_encode(epad, ea, w_ee0, b_of(ee0), w_of(ee1), b_of(ee1),
                          wle, ble)

    sc_layer = _sc_layer_fn(npad, epad, nsc)
    deg_parts = _sc_deg_fn(npad, epad)(dst)
    d0, d1 = deg_parts[0], deg_parts[1]

    for l in range(NLAYERS):
        lp = layers[l]
        parts = sc_layer(hs, hd, ep_all[l], src, dst)
        has_next = l + 1 < NLAYERS
        nxt = layers[l + 1] if has_next else layers[0]
        outs = _node_update(
            npad, has_next, h, parts[0], parts[1], d0, d1,
            w_of(lp["msg2"]), b_of(lp["msg2"]),
            w_of(lp["node1"])[0:H], w_of(lp["node1"])[H:2 * H],
            b_of(lp["node1"]), w_of(lp["node2"]), b_of(lp["node2"]),
            lp["ln_g"].reshape(1, H), lp["ln_b"].reshape(1, H),
            wls[(l + 1) % NLAYERS], wld[(l + 1) % NLAYERS])
        if has_next:
            h, hs, hd = outs
        else:
            h = outs[0]

    d0p, d1p, d2p = pr["dec"]
    w_d2 = jnp.pad(w_of(d1p), ((0, 0), (0, H - 64)))
    b_d2 = jnp.pad(b_of(d1p), ((0, 0), (0, H - 64)))
    w_d3 = jnp.pad(w_of(d2p), ((0, H - 64), (0, H - 3)))
    b_d3 = jnp.pad(b_of(d2p), ((0, 0), (0, H - 3)))
    out = _decode(npad, h, w_of(d0p), b_of(d0p), w_d2, b_d2, w_d3, b_d3)

    pred = out[:n, :3]
    mask = jnp.concatenate([1.0 - bc_disp, 1.0 - bc_disp, 1.0 - bc_rot],
                           axis=-1)
    return pred * mask
